# baseline (speedup 1.0000x reference)
import functools

import jax
import jax.numpy as jnp
from jax import lax
from jax.experimental import pallas as pl
from jax.experimental.pallas import tpu as pltpu
from jax.experimental.pallas import tpu_sc as plsc

EMBED = 128
CTX = 2
LANES = 16
NJ = EMBED // LANES


def _predictor_sc(y3, table, w0, w1, *, B, U):
    V = table.shape[0]
    info = plsc.get_sparse_core_info()
    NC, NS = info.num_cores, info.num_subcores
    NW = NC * NS
    nb = B // NW
    HALF = U // 2
    NBUF = 2
    nbg = nb // NBUF

    mesh = plsc.VectorSubcoreMesh(core_axis_name="c", subcore_axis_name="s")

    @functools.partial(
        pl.kernel,
        out_type=jax.ShapeDtypeStruct((B, U, EMBED), jnp.float32),
        mesh=mesh,
        scratch_types=[
            pltpu.VMEM((nb, CTX, HALF), jnp.int32),
            pltpu.VMEM((NBUF, U, EMBED), jnp.float32),
            pltpu.VMEM((NBUF, U, EMBED), jnp.float32),
            pltpu.VMEM_SHARED((V, EMBED), jnp.float32),
            pltpu.VMEM((EMBED,), jnp.float32),
            pltpu.VMEM((EMBED,), jnp.float32),
            [pltpu.SemaphoreType.DMA] * NBUF,
            [pltpu.SemaphoreType.DMA] * NBUF,
        ],
    )
    def body(y_hbm, table_hbm, w0_hbm, w1_hbm, out_hbm,
             idx_v, rows_v, outb_v, table_sp, w0_v, w1_v, gsem, osem):
        wid = lax.axis_index("s") * NC + lax.axis_index("c")
        base = wid * nb

        @pl.when(lax.axis_index("s") == 0)
        def _():
            pltpu.sync_copy(table_hbm, table_sp)
        pltpu.sync_copy(w0_hbm, w0_v)
        pltpu.sync_copy(w1_hbm, w1_v)
        pltpu.sync_copy(y_hbm.at[pl.ds(base, nb)], idx_v)
        plsc.subcore_barrier()

        w0r = [w0_v[pl.ds(LANES * j, LANES)] for j in range(NJ)]
        w1r = [w1_v[pl.ds(LANES * j, LANES)] for j in range(NJ)]

        def gather_descs(i, buf):
            return [
                pltpu.make_async_copy(
                    table_sp.at[idx_v.at[i, h]],
                    rows_v.at[buf, pl.ds(h * HALF, HALF)],
                    gsem[buf],
                )
                for h in range(CTX)
            ]

        def start_gather(i, buf):
            for d in gather_descs(i, buf):
                d.start()

        def wait_gather(i, buf):
            for d in gather_descs(i, buf):
                d.wait()

        def out_desc(i, buf):
            return pltpu.make_async_copy(
                outb_v.at[buf], out_hbm.at[base + i], osem[buf])

        def compute(buf):
            zero = tuple(jnp.zeros((LANES,), jnp.float32) for _ in range(NJ))

            @plsc.parallel_loop(0, U, 1, unroll=4, carry=zero)
            def _(u, carry):
                nxt = []
                for j in range(NJ):
                    t = rows_v[buf, u, pl.ds(LANES * j, LANES)]
                    outb_v[buf, u, pl.ds(LANES * j, LANES)] = (
                        w1r[j] * t + w0r[j] * carry[j])
                    nxt.append(t)
                return tuple(nxt)

        start_gather(0, 0)

        def gbody(g, _):
            for b in range(NBUF):
                i = g * NBUF + b
                wait_gather(i, b)
                if b == NBUF - 1:
                    @pl.when(g < nbg - 1)
                    def _():
                        start_gather(i + 1, 0)
                else:
                    start_gather(i + 1, b + 1)
                @pl.when(g >= 1)
                def _():
                    out_desc(i - NBUF, b).wait()
                compute(b)
                out_desc(i, b).start()
            return 0

        lax.fori_loop(0, nbg, gbody, 0)
        for b in range(NBUF):
            out_desc(nb - NBUF + b, b).wait()

    return body


def kernel(y, table, conv_w):
    B, U = y.shape
    y3 = y.astype(jnp.int32).reshape(B, CTX, U // CTX)
    w0 = conv_w[:, 0, 0]
    w1 = conv_w[:, 0, 1]
    return _predictor_sc(y3, table, w0, w1, B=B, U=U)(y3, table, w0, w1)


# speedup vs baseline: 1.0202x; 1.0202x over previous
import functools

import jax
import jax.numpy as jnp
from jax import lax
from jax.experimental import pallas as pl
from jax.experimental.pallas import tpu as pltpu
from jax.experimental.pallas import tpu_sc as plsc

EMBED = 128
CTX = 2
LANES = 16
NJ = EMBED // LANES


def _predictor_sc(y3, table, w0, w1, *, B, U):
    V = table.shape[0]
    info = plsc.get_sparse_core_info()
    NC, NS = info.num_cores, info.num_subcores
    NW = NC * NS
    nb = B // NW
    HALF = U // 2
    NBUF = 4
    PF = 2
    nbg = nb // NBUF

    mesh = plsc.VectorSubcoreMesh(core_axis_name="c", subcore_axis_name="s")

    @functools.partial(
        pl.kernel,
        out_type=jax.ShapeDtypeStruct((B, U, EMBED), jnp.float32),
        mesh=mesh,
        scratch_types=[
            pltpu.VMEM((nb, CTX, HALF), jnp.int32),
            pltpu.VMEM((NBUF, U, EMBED), jnp.float32),
            pltpu.VMEM_SHARED((V, EMBED), jnp.float32),
            pltpu.VMEM((EMBED,), jnp.float32),
            pltpu.VMEM((EMBED,), jnp.float32),
            [pltpu.SemaphoreType.DMA] * NBUF,
            [pltpu.SemaphoreType.DMA] * NBUF,
        ],
    )
    def body(y_hbm, table_hbm, w0_hbm, w1_hbm, out_hbm,
             idx_v, rows_v, table_sp, w0_v, w1_v, gsem, osem):
        wid = lax.axis_index("s") * NC + lax.axis_index("c")
        base = wid * nb

        @pl.when(lax.axis_index("s") == 0)
        def _():
            pltpu.sync_copy(table_hbm, table_sp)
        pltpu.sync_copy(w0_hbm, w0_v)
        pltpu.sync_copy(w1_hbm, w1_v)
        pltpu.sync_copy(y_hbm.at[pl.ds(base, nb)], idx_v)
        plsc.subcore_barrier()
        w0r = [w0_v[pl.ds(LANES * j, LANES)] for j in range(NJ)]
        w1r = [w1_v[pl.ds(LANES * j, LANES)] for j in range(NJ)]

        def gather_descs(i, buf):
            return [
                pltpu.make_async_copy(
                    table_sp.at[idx_v.at[i, h]],
                    rows_v.at[buf, pl.ds(h * HALF, HALF)],
                    gsem[buf],
                )
                for h in range(CTX)
            ]

        def start_gather(i, buf):
            for d in gather_descs(i, buf):
                d.start()

        def wait_gather(i, buf):
            for d in gather_descs(i, buf):
                d.wait()

        def out_desc(i, buf):
            return pltpu.make_async_copy(
                rows_v.at[buf], out_hbm.at[base + i], osem[buf])

        def compute(buf):
            carry0 = tuple(
                rows_v[buf, U - 1, pl.ds(LANES * j, LANES)] for j in range(NJ))

            def ubody(step, carry):
                u = (U - 1) - step
                nxt = []
                for j in range(NJ):
                    t = rows_v[buf, u - 1, pl.ds(LANES * j, LANES)]
                    rows_v[buf, u, pl.ds(LANES * j, LANES)] = (
                        w1r[j] * carry[j] + w0r[j] * t)
                    nxt.append(t)
                return tuple(nxt)

            last = lax.fori_loop(0, U - 1, ubody, carry0)
            for j in range(NJ):
                rows_v[buf, 0, pl.ds(LANES * j, LANES)] = w1r[j] * last[j]

        for k in range(PF):
            start_gather(k, k)

        def gbody(g, _):
            for b in range(NBUF):
                i = g * NBUF + b
                nxt_buf = (b + PF) % NBUF
                if b < PF:
                    @pl.when(g >= 1)
                    def _():
                        out_desc(i - PF, nxt_buf).wait()
                    start_gather(i + PF, nxt_buf)
                else:
                    out_desc(i - PF, nxt_buf).wait()
                    @pl.when(g < nbg - 1)
                    def _():
                        start_gather(i + PF, nxt_buf)
                wait_gather(i, b)
                compute(b)
                out_desc(i, b).start()
            return 0

        lax.fori_loop(0, nbg, gbody, 0)
        for b in range(NBUF - PF, NBUF):
            out_desc(nb - NBUF + b, b).wait()

    return body


def kernel(y, table, conv_w):
    B, U = y.shape
    y3 = y.astype(jnp.int32).reshape(B, CTX, U // CTX)
    w0 = conv_w[:, 0, 0]
    w1 = conv_w[:, 0, 1]
    return _predictor_sc(y3, table, w0, w1, B=B, U=U)(y3, table, w0, w1)
